# baseline (speedup 1.0000x reference)
import jax
import jax.numpy as jnp
from jax.experimental import pallas as pl
from jax.experimental.pallas import tpu as pltpu

_PREC = jax.lax.Precision.DEFAULT


def _dot(a, b, dims):
    return jax.lax.dot_general(a, b, (dims, ((), ())), precision=_PREC,
                               preferred_element_type=jnp.float32)


def _mm(a, b):
    return _dot(a, b, ((1,), (0,)))


def _mmb(a, b):
    return _dot(a.astype(jnp.bfloat16), b, ((1,), (0,)))


def _cheb_kernel(x0_ref, l_ref, w1_ref, b1_ref, w2_ref, b2_ref, eye_ref,
                 out_ref, s_ref, acc_ref):
    j = pl.program_id(0)
    kk1, g1 = w1_ref.shape[0], w1_ref.shape[1]
    b, n = x0_ref.shape
    ncheb = 5
    tcols = out_ref.shape[0]

    def contrib_pair(a0, a1):
        cat = jnp.concatenate(
            [a0.reshape(g1, b, n).reshape(g1, b * n),
             a1.reshape(g1, b, n).reshape(g1, b * n)], axis=0)
        w2p = w2_ref[...].reshape(2 * g1, w2_ref.shape[2])
        return _dot(w2p, cat, ((0,), (0,)))

    @pl.when(j == 0)
    def _():
        lmat = l_ref[...].astype(jnp.bfloat16)
        t_pp = x0_ref[...]
        t_p = _mmb(t_pp, lmat)
        h1 = w1_ref[0] * t_pp[None] + w1_ref[1] * t_p[None]
        for i in range(2, kk1):
            t_new = 2.0 * _mmb(t_p, lmat) - t_pp
            h1 = h1 + w1_ref[i] * t_new[None]
            t_pp, t_p = t_p, t_new
        a0 = jnp.maximum(h1 + b1_ref[...], 0.0).reshape(g1 * b, n)
        a1 = _mmb(a0, lmat)
        s_ref[0] = a0
        s_ref[1] = a1
        acc_ref[...] = contrib_pair(a0, a1)

    @pl.when((j > 0) & (j < ncheb))
    def _():
        lmat = l_ref[...].astype(jnp.bfloat16)
        a_lo = 2.0 * _mmb(s_ref[1], lmat) - s_ref[0]
        a_hi = 2.0 * _mmb(a_lo, lmat) - s_ref[1]
        s_ref[0] = a_lo
        s_ref[1] = a_hi
        acc_ref[...] = acc_ref[...] + contrib_pair(a_lo, a_hi)

    for cc in range((b * n) // tcols):
        @pl.when(j == ncheb + cc)
        def _(cc=cc):
            sl = jnp.maximum(
                acc_ref[:, cc * tcols:(cc + 1) * tcols] + b2_ref[...], 0.0)
            out_ref[...] = _dot(sl, eye_ref[...],
                                ((0,), (0,))).astype(jnp.bfloat16)


def _fc_kernel(h_ref, w_ref, fb_ref, v_ref, vb_ref, o_ref, acc_ref):
    i = pl.program_id(0)
    p = _dot(h_ref[...], w_ref[...].astype(jnp.bfloat16), ((1,), (0,)))

    @pl.when(i == 0)
    def _():
        acc_ref[...] = p

    @pl.when(i > 0)
    def _():
        acc_ref[...] = acc_ref[...] + p

    @pl.when(i == pl.num_programs(0) - 1)
    def _():
        h = jnp.maximum(acc_ref[...] + fb_ref[...], 0.0)
        logits = _mm(h, v_ref[...]) + vb_ref[...]
        s = logits - jnp.max(logits, axis=1, keepdims=True)
        o_ref[...] = s - jnp.log(jnp.sum(jnp.exp(s), axis=1, keepdims=True))


def kernel(x, L, W1, b1, W2, b2, fc1_W, fc1_b, fc2_W, fc2_b):
    bsz, n, _ = x.shape
    k1, _, g1 = W1.shape
    k2, _, g2 = W2.shape
    d = fc1_W.shape[1]
    c = fc2_W.shape[1]

    x0 = x[:, :, 0]
    w1b = W1[:, 0, :, None, None]
    b1b = b1[:, None, None]
    eye = jnp.eye(g2, dtype=jnp.float32)

    ntr = 4
    tcols = (bsz * n) // ntr
    ncheb = k2 // 2
    h2f = pl.pallas_call(
        _cheb_kernel,
        grid=(ncheb + ntr,),
        in_specs=[
            pl.BlockSpec((bsz, n), lambda j: (0, 0)),
            pl.BlockSpec((n, n), lambda j: (0, 0)),
            pl.BlockSpec((k1, g1, 1, 1), lambda j: (0, 0, 0, 0)),
            pl.BlockSpec((g1, 1, 1), lambda j: (0, 0, 0)),
            pl.BlockSpec((2, g1, g2),
                         lambda j: (jnp.minimum(j, 4), 0, 0)),
            pl.BlockSpec((g2, 1), lambda j: (0, 0)),
            pl.BlockSpec((g2, g2), lambda j: (0, 0)),
        ],
        out_specs=pl.BlockSpec((tcols, g2),
                               lambda j: (jnp.maximum(j - 5, 0), 0)),
        out_shape=jax.ShapeDtypeStruct((bsz * n, g2), jnp.bfloat16),
        scratch_shapes=[
            pltpu.VMEM((2, g1 * bsz, n), jnp.float32),
            pltpu.VMEM((g2, bsz * n), jnp.float32),
        ],
    )(x0, L, w1b, b1b, W2, b2[:, None], eye)

    h2 = h2f.reshape(bsz, n * g2)
    nblk = 16
    chw = (n * g2) // nblk
    out = pl.pallas_call(
        _fc_kernel,
        grid=(nblk,),
        in_specs=[
            pl.BlockSpec((bsz, chw), lambda i: (0, i)),
            pl.BlockSpec((chw, d), lambda i: (i, 0)),
            pl.BlockSpec((1, d), lambda i: (0, 0)),
            pl.BlockSpec((d, c), lambda i: (0, 0)),
            pl.BlockSpec((1, c), lambda i: (0, 0)),
        ],
        out_specs=pl.BlockSpec((bsz, c), lambda i: (0, 0)),
        out_shape=jax.ShapeDtypeStruct((bsz, c), jnp.float32),
        scratch_shapes=[pltpu.VMEM((bsz, d), jnp.float32)],
    )(h2, fc1_W, fc1_b[None, :], fc2_W, fc2_b[None, :])
    return out


# speedup vs baseline: 1.0006x; 1.0006x over previous
import jax
import jax.numpy as jnp
from jax.experimental import pallas as pl
from jax.experimental.pallas import tpu as pltpu

_PREC = jax.lax.Precision.DEFAULT


def _dot(a, b, dims):
    return jax.lax.dot_general(a, b, (dims, ((), ())), precision=_PREC,
                               preferred_element_type=jnp.float32)


def _mm(a, b):
    return _dot(a, b, ((1,), (0,)))


def _mmb(a, b):
    return _dot(a.astype(jnp.bfloat16), b, ((1,), (0,)))


def _cheb_kernel(x0_ref, l_ref, w1_ref, b1_ref, w2_ref, b2_ref, eye_ref,
                 out_ref, s_ref, acc_ref, lb_ref):
    j = pl.program_id(0)
    kk1, g1 = w1_ref.shape[0], w1_ref.shape[1]
    b, n = x0_ref.shape
    ncheb = 5
    tcols = out_ref.shape[0]

    def contrib_pair(a0, a1):
        cat = jnp.concatenate(
            [a0.reshape(g1, b, n).reshape(g1, b * n),
             a1.reshape(g1, b, n).reshape(g1, b * n)], axis=0)
        w2p = w2_ref[...].reshape(2 * g1, w2_ref.shape[2])
        return _dot(w2p, cat, ((0,), (0,)))

    @pl.when(j == 0)
    def _():
        lmat = l_ref[...].astype(jnp.bfloat16)
        lb_ref[...] = 2.0 * lmat
        t_pp = x0_ref[...]
        t_p = _mmb(t_pp, lmat)
        h1 = w1_ref[0] * t_pp[None] + w1_ref[1] * t_p[None]
        for i in range(2, kk1):
            t_new = 2.0 * _mmb(t_p, lmat) - t_pp
            h1 = h1 + w1_ref[i] * t_new[None]
            t_pp, t_p = t_p, t_new
        a0 = jnp.maximum(h1 + b1_ref[...], 0.0).reshape(g1 * b, n)
        a1 = _mmb(a0, lmat)
        s_ref[0] = a0
        s_ref[1] = a1
        acc_ref[...] = contrib_pair(a0, a1)

    @pl.when((j > 0) & (j < ncheb))
    def _():
        lmat2 = lb_ref[...]
        a_lo = _mmb(s_ref[1], lmat2) - s_ref[0]
        a_hi = _mmb(a_lo, lmat2) - s_ref[1]
        s_ref[0] = a_lo
        s_ref[1] = a_hi
        acc_ref[...] = acc_ref[...] + contrib_pair(a_lo, a_hi)

    for cc in range((b * n) // tcols):
        @pl.when(j == ncheb + cc)
        def _(cc=cc):
            sl = jnp.maximum(
                acc_ref[:, cc * tcols:(cc + 1) * tcols] + b2_ref[...], 0.0)
            out_ref[...] = _dot(sl, eye_ref[...],
                                ((0,), (0,))).astype(jnp.bfloat16)


def _fc_kernel(h_ref, w_ref, fb_ref, v_ref, vb_ref, o_ref, acc_ref):
    i = pl.program_id(0)
    p = _dot(h_ref[...], w_ref[...].astype(jnp.bfloat16), ((1,), (0,)))

    @pl.when(i == 0)
    def _():
        acc_ref[...] = p

    @pl.when(i > 0)
    def _():
        acc_ref[...] = acc_ref[...] + p

    @pl.when(i == pl.num_programs(0) - 1)
    def _():
        h = jnp.maximum(acc_ref[...] + fb_ref[...], 0.0)
        logits = _mm(h, v_ref[...]) + vb_ref[...]
        s = logits - jnp.max(logits, axis=1, keepdims=True)
        o_ref[...] = s - jnp.log(jnp.sum(jnp.exp(s), axis=1, keepdims=True))


def kernel(x, L, W1, b1, W2, b2, fc1_W, fc1_b, fc2_W, fc2_b):
    bsz, n, _ = x.shape
    k1, _, g1 = W1.shape
    k2, _, g2 = W2.shape
    d = fc1_W.shape[1]
    c = fc2_W.shape[1]

    x0 = x[:, :, 0]
    w1b = W1[:, 0, :, None, None]
    b1b = b1[:, None, None]
    eye = jnp.eye(g2, dtype=jnp.float32)

    ntr = 4
    tcols = (bsz * n) // ntr
    ncheb = k2 // 2
    h2f = pl.pallas_call(
        _cheb_kernel,
        grid=(ncheb + ntr,),
        in_specs=[
            pl.BlockSpec((bsz, n), lambda j: (0, 0)),
            pl.BlockSpec((n, n), lambda j: (0, 0)),
            pl.BlockSpec((k1, g1, 1, 1), lambda j: (0, 0, 0, 0)),
            pl.BlockSpec((g1, 1, 1), lambda j: (0, 0, 0)),
            pl.BlockSpec((2, g1, g2),
                         lambda j: (jnp.minimum(j, 4), 0, 0)),
            pl.BlockSpec((g2, 1), lambda j: (0, 0)),
            pl.BlockSpec((g2, g2), lambda j: (0, 0)),
        ],
        out_specs=pl.BlockSpec((tcols, g2),
                               lambda j: (jnp.maximum(j - 5, 0), 0)),
        out_shape=jax.ShapeDtypeStruct((bsz * n, g2), jnp.bfloat16),
        scratch_shapes=[
            pltpu.VMEM((2, g1 * bsz, n), jnp.float32),
            pltpu.VMEM((g2, bsz * n), jnp.float32),
            pltpu.VMEM((n, n), jnp.bfloat16),
        ],
    )(x0, L, w1b, b1b, W2, b2[:, None], eye)

    h2 = h2f.reshape(bsz, n * g2)
    nblk = 16
    chw = (n * g2) // nblk
    out = pl.pallas_call(
        _fc_kernel,
        grid=(nblk,),
        in_specs=[
            pl.BlockSpec((bsz, chw), lambda i: (0, i)),
            pl.BlockSpec((chw, d), lambda i: (i, 0)),
            pl.BlockSpec((1, d), lambda i: (0, 0)),
            pl.BlockSpec((d, c), lambda i: (0, 0)),
            pl.BlockSpec((1, c), lambda i: (0, 0)),
        ],
        out_specs=pl.BlockSpec((bsz, c), lambda i: (0, 0)),
        out_shape=jax.ShapeDtypeStruct((bsz, c), jnp.float32),
        scratch_shapes=[pltpu.VMEM((bsz, d), jnp.float32)],
    )(h2, fc1_W, fc1_b[None, :], fc2_W, fc2_b[None, :])
    return out
